# baseline (speedup 1.0000x reference)
import functools

import jax
import jax.numpy as jnp
from jax import lax
from jax.experimental import pallas as pl
from jax.experimental.pallas import tpu as pltpu
from jax.experimental.pallas import tpu_sc as plsc

N = 10000
E = 320000
D = 128
H = 128
C = 10

NP = 10240
NC = 2
NS = 16
NW = NC * NS
CH = 128
EP = NW * CH * ((E + NW * CH - 1) // (NW * CH))
EPT = EP // NW
NCHK = EPT // CH
RPT = NP // NS


def _deg_body(idx_hbm, ones_hbm, zero_hbm, deg_out, slot_v, ones_v, acc_sh):
    cid = lax.axis_index("c")
    sid = lax.axis_index("s")
    wid = cid * NS + sid
    rbase = sid * RPT
    pltpu.sync_copy(zero_hbm.at[pl.ds(rbase, RPT)], acc_sh.at[pl.ds(rbase, RPT)])
    pltpu.sync_copy(ones_hbm, ones_v)
    plsc.subcore_barrier()

    def body(j, carry):
        pltpu.sync_copy(idx_hbm.at[wid, j], slot_v)
        pltpu.sync_copy(ones_v, acc_sh.at[slot_v.at[1]], add=True)
        return carry

    lax.fori_loop(0, NCHK, body, 0)
    plsc.subcore_barrier()
    pltpu.sync_copy(acc_sh.at[pl.ds(rbase, RPT)],
                    deg_out.at[cid, pl.ds(rbase, RPT)])


def _agg_body(hs_hbm, idx_hbm, zrows_hbm, agg_out,
              idxall_v, gath0_v, acc_sh, gsem0):
    cid = lax.axis_index("c")
    sid = lax.axis_index("s")
    wid = cid * NS + sid
    rbase = sid * RPT
    pltpu.sync_copy(zrows_hbm.at[pl.ds(rbase, RPT)],
                    acc_sh.at[pl.ds(rbase, RPT)])
    pltpu.sync_copy(idx_hbm.at[wid], idxall_v)
    plsc.subcore_barrier()

    def body(j, carry):
        pltpu.async_copy(hs_hbm.at[idxall_v.at[j, 0]], gath0_v, gsem0).wait()
        pltpu.sync_copy(gath0_v, acc_sh.at[idxall_v.at[j, 1]], add=True)
        return carry

    lax.fori_loop(0, NCHK, body, 0)
    plsc.subcore_barrier()
    pltpu.sync_copy(acc_sh.at[pl.ds(rbase, RPT), :],
                    agg_out.at[cid, pl.ds(rbase, RPT), :])


@functools.cache
def _sc_kernels():
    mesh = plsc.VectorSubcoreMesh(core_axis_name="c", subcore_axis_name="s")
    deg_k = pl.kernel(
        _deg_body,
        mesh=mesh,
        out_type=jax.ShapeDtypeStruct((NC, NP), jnp.float32),
        scratch_types=[
            pltpu.VMEM((2, CH), jnp.int32),
            pltpu.VMEM((CH,), jnp.float32),
            pltpu.VMEM_SHARED((NP,), jnp.float32),
        ],
    )
    agg_k = pl.kernel(
        _agg_body,
        mesh=mesh,
        out_type=jax.ShapeDtypeStruct((NC, NP, H), jnp.float32),
        scratch_types=[
            pltpu.VMEM((NCHK, 2, CH), jnp.int32),
            pltpu.VMEM((CH, H), jnp.float32),
            pltpu.VMEM_SHARED((NP, H), jnp.float32),
            pltpu.SemaphoreType.DMA,
        ],
    )
    return deg_k, agg_k


_RB = 1000


def _tc_in_body(deg_ref, x_ref, w_ref, hs_ref, dinv_ref):
    dd = deg_ref[0] + deg_ref[1] + 1.0
    dinv = lax.rsqrt(dd)
    h = jnp.dot(x_ref[...], w_ref[...], preferred_element_type=jnp.float32)
    hs_ref[...] = h * dinv
    dinv_ref[...] = dinv


def _tc_mid_body(agg_ref, hp_ref, dinv_ref, b_ref, w_ref, out_ref):
    dinv = dinv_ref[...]
    pre = (agg_ref[0] + agg_ref[1] + hp_ref[...]) * dinv + b_ref[...]
    act = jnp.maximum(pre, 0.0)
    out_ref[...] = jnp.dot(act, w_ref[...],
                           preferred_element_type=jnp.float32) * dinv


def _tc_out_body(agg_ref, hp_ref, dinv_ref, b_ref, wc_ref, bc_ref, out_ref):
    dinv = dinv_ref[...]
    pre = (agg_ref[0] + agg_ref[1] + hp_ref[...]) * dinv + b_ref[...]
    act = jnp.maximum(pre, 0.0)
    out_ref[...] = jnp.dot(act, wc_ref[...],
                           preferred_element_type=jnp.float32) + bc_ref[...]


def _tc_in(deg3, x, w):
    return pl.pallas_call(
        _tc_in_body,
        grid=(N // _RB,),
        in_specs=[
            pl.BlockSpec((NC, _RB, 1), lambda i: (0, i, 0)),
            pl.BlockSpec((_RB, D), lambda i: (i, 0)),
            pl.BlockSpec((D, H), lambda i: (0, 0)),
        ],
        out_specs=[
            pl.BlockSpec((_RB, H), lambda i: (i, 0)),
            pl.BlockSpec((_RB, 1), lambda i: (i, 0)),
        ],
        out_shape=[
            jax.ShapeDtypeStruct((N, H), jnp.float32),
            jax.ShapeDtypeStruct((N, 1), jnp.float32),
        ],
    )(deg3, x, w)


def _tc_mid(agg, hp, dinv, b2d, w):
    return pl.pallas_call(
        _tc_mid_body,
        grid=(N // _RB,),
        in_specs=[
            pl.BlockSpec((NC, _RB, H), lambda i: (0, i, 0)),
            pl.BlockSpec((_RB, H), lambda i: (i, 0)),
            pl.BlockSpec((_RB, 1), lambda i: (i, 0)),
            pl.BlockSpec((1, H), lambda i: (0, 0)),
            pl.BlockSpec((H, H), lambda i: (0, 0)),
        ],
        out_specs=pl.BlockSpec((_RB, H), lambda i: (i, 0)),
        out_shape=jax.ShapeDtypeStruct((N, H), jnp.float32),
    )(agg, hp, dinv, b2d, w)


def _tc_out(agg, hp, dinv, b2d, wc_pad, bc_pad):
    return pl.pallas_call(
        _tc_out_body,
        grid=(N // _RB,),
        in_specs=[
            pl.BlockSpec((NC, _RB, H), lambda i: (0, i, 0)),
            pl.BlockSpec((_RB, H), lambda i: (i, 0)),
            pl.BlockSpec((_RB, 1), lambda i: (i, 0)),
            pl.BlockSpec((1, H), lambda i: (0, 0)),
            pl.BlockSpec((H, H), lambda i: (0, 0)),
            pl.BlockSpec((1, H), lambda i: (0, 0)),
        ],
        out_specs=pl.BlockSpec((_RB, H), lambda i: (i, 0)),
        out_shape=jax.ShapeDtypeStruct((N, H), jnp.float32),
    )(agg, hp, dinv, b2d, wc_pad, bc_pad)


def kernel(x, edge_index, W1, b1, W2, b2, W3, b3, Wc, bc):
    pad = EP - E
    src_p = jnp.concatenate([edge_index[0], jnp.zeros((pad,), jnp.int32)])
    dst_p = jnp.concatenate(
        [edge_index[1],
         (N + jnp.arange(pad, dtype=jnp.int32) % (NP - N)).astype(jnp.int32)])
    idx2 = jnp.stack([src_p.reshape(NW, NCHK, CH),
                      dst_p.reshape(NW, NCHK, CH)], axis=2)
    ones_ch = jnp.ones((CH,), jnp.float32)
    zero_n = jnp.zeros((NP,), jnp.float32)
    zrows = jnp.zeros((NP, H), jnp.float32)

    deg_kernel, agg_kernel = _sc_kernels()
    deg = deg_kernel(idx2, ones_ch, zero_n)
    deg3 = deg[:, :N].reshape(NC, N, 1)

    hs1, dinv = _tc_in(deg3, x, W1)
    agg1 = agg_kernel(hs1, idx2, zrows)[:, :N, :]
    hs2 = _tc_mid(agg1, hs1, dinv, b1.reshape(1, H), W2)
    agg2 = agg_kernel(hs2, idx2, zrows)[:, :N, :]
    hs3 = _tc_mid(agg2, hs2, dinv, b2.reshape(1, H), W3)
    agg3 = agg_kernel(hs3, idx2, zrows)[:, :N, :]

    wc_pad = jnp.zeros((H, H), jnp.float32).at[:, :C].set(Wc)
    bc_pad = jnp.zeros((1, H), jnp.float32).at[0, :C].set(bc)
    out = _tc_out(agg3, hs3, dinv, b3.reshape(1, H), wc_pad, bc_pad)
    return out[:, :C]


# speedup vs baseline: 1.0904x; 1.0904x over previous
import functools

import jax
import jax.numpy as jnp
from jax import lax
from jax.experimental import pallas as pl
from jax.experimental.pallas import tpu as pltpu
from jax.experimental.pallas import tpu_sc as plsc

N = 10000
E = 320000
D = 128
H = 128
C = 10

NP = 10240
NC = 2
NS = 16
NW = NC * NS
CH = 128
EP = NW * CH * ((E + NW * CH - 1) // (NW * CH))
EPT = EP // NW
NCHK = EPT // CH
RPT = NP // NS


def _deg_body(idx_hbm, ones_hbm, zero_hbm, deg_out, slot_v, ones_v, acc_sh):
    cid = lax.axis_index("c")
    sid = lax.axis_index("s")
    wid = cid * NS + sid
    rbase = sid * RPT
    pltpu.sync_copy(zero_hbm.at[pl.ds(rbase, RPT)], acc_sh.at[pl.ds(rbase, RPT)])
    pltpu.sync_copy(ones_hbm, ones_v)
    plsc.subcore_barrier()

    def body(j, carry):
        pltpu.sync_copy(idx_hbm.at[wid, j], slot_v)
        pltpu.sync_copy(ones_v, acc_sh.at[slot_v.at[1]], add=True)
        return carry

    lax.fori_loop(0, NCHK, body, 0)
    plsc.subcore_barrier()
    pltpu.sync_copy(acc_sh.at[pl.ds(rbase, RPT)],
                    deg_out.at[cid, pl.ds(rbase, RPT)])


def _agg_body(hs_hbm, idx_hbm, zrows_hbm, agg_out,
              idxall_v, gath0_v, acc_sh, gsem0):
    cid = lax.axis_index("c")
    sid = lax.axis_index("s")
    wid = cid * NS + sid
    rbase = sid * RPT
    pltpu.sync_copy(zrows_hbm.at[pl.ds(rbase, RPT)],
                    acc_sh.at[pl.ds(rbase, RPT)])
    pltpu.sync_copy(idx_hbm.at[wid], idxall_v)
    plsc.subcore_barrier()

    def body(j, carry):
        pltpu.async_copy(hs_hbm.at[idxall_v.at[j, 0]], gath0_v, gsem0).wait()
        pltpu.sync_copy(gath0_v, acc_sh.at[idxall_v.at[j, 1]], add=True)
        return carry

    lax.fori_loop(0, NCHK, body, 0)
    plsc.subcore_barrier()
    pltpu.sync_copy(acc_sh.at[pl.ds(rbase, RPT), :],
                    agg_out.at[cid, pl.ds(rbase, RPT), :])


@functools.cache
def _sc_kernels():
    mesh = plsc.VectorSubcoreMesh(core_axis_name="c", subcore_axis_name="s")
    deg_k = pl.kernel(
        _deg_body,
        mesh=mesh,
        out_type=jax.ShapeDtypeStruct((NC, NP), jnp.float32),
        scratch_types=[
            pltpu.VMEM((2, CH), jnp.int32),
            pltpu.VMEM((CH,), jnp.float32),
            pltpu.VMEM_SHARED((NP,), jnp.float32),
        ],
    )
    agg_k = pl.kernel(
        _agg_body,
        mesh=mesh,
        out_type=jax.ShapeDtypeStruct((NC, NP, H), jnp.float32),
        scratch_types=[
            pltpu.VMEM((NCHK, 2, CH), jnp.int32),
            pltpu.VMEM((CH, H), jnp.float32),
            pltpu.VMEM_SHARED((NP, H), jnp.float32),
            pltpu.SemaphoreType.DMA,
        ],
    )
    return deg_k, agg_k


_RB = 1000


def _tc_in_body(deg_ref, x_ref, w_ref, hs_ref, dinv_ref):
    dd = deg_ref[0] + deg_ref[1] + 1.0
    dinv = lax.rsqrt(dd)
    h = jnp.dot(x_ref[...], w_ref[...], preferred_element_type=jnp.float32)
    hs_ref[...] = h * dinv
    dinv_ref[...] = dinv


def _tc_mid_body(agg_ref, hp_ref, dinv_ref, b_ref, w_ref, out_ref):
    dinv = dinv_ref[...]
    pre = (agg_ref[0] + agg_ref[1] + hp_ref[...]) * dinv + b_ref[...]
    act = jnp.maximum(pre, 0.0)
    out_ref[...] = jnp.dot(act, w_ref[...],
                           preferred_element_type=jnp.float32) * dinv


def _tc_out_body(agg_ref, hp_ref, dinv_ref, b_ref, wc_ref, bc_ref, out_ref):
    dinv = dinv_ref[...]
    pre = (agg_ref[0] + agg_ref[1] + hp_ref[...]) * dinv + b_ref[...]
    act = jnp.maximum(pre, 0.0)
    out_ref[...] = jnp.dot(act, wc_ref[...],
                           preferred_element_type=jnp.float32) + bc_ref[...]


def _tc_in(deg3, x, w):
    return pl.pallas_call(
        _tc_in_body,
        grid=(N // _RB,),
        in_specs=[
            pl.BlockSpec((NC, _RB, 1), lambda i: (0, i, 0)),
            pl.BlockSpec((_RB, D), lambda i: (i, 0)),
            pl.BlockSpec((D, H), lambda i: (0, 0)),
        ],
        out_specs=[
            pl.BlockSpec((_RB, H), lambda i: (i, 0)),
            pl.BlockSpec((_RB, 1), lambda i: (i, 0)),
        ],
        out_shape=[
            jax.ShapeDtypeStruct((N, H), jnp.float32),
            jax.ShapeDtypeStruct((N, 1), jnp.float32),
        ],
    )(deg3, x, w)


def _tc_mid(agg, hp, dinv, b2d, w):
    return pl.pallas_call(
        _tc_mid_body,
        grid=(N // _RB,),
        in_specs=[
            pl.BlockSpec((NC, _RB, H), lambda i: (0, i, 0)),
            pl.BlockSpec((_RB, H), lambda i: (i, 0)),
            pl.BlockSpec((_RB, 1), lambda i: (i, 0)),
            pl.BlockSpec((1, H), lambda i: (0, 0)),
            pl.BlockSpec((H, H), lambda i: (0, 0)),
        ],
        out_specs=pl.BlockSpec((_RB, H), lambda i: (i, 0)),
        out_shape=jax.ShapeDtypeStruct((N, H), jnp.float32),
    )(agg, hp, dinv, b2d, w)


def _tc_out(agg, hp, dinv, b2d, wc_pad, bc_pad):
    return pl.pallas_call(
        _tc_out_body,
        grid=(N // _RB,),
        in_specs=[
            pl.BlockSpec((NC, _RB, H), lambda i: (0, i, 0)),
            pl.BlockSpec((_RB, H), lambda i: (i, 0)),
            pl.BlockSpec((_RB, 1), lambda i: (i, 0)),
            pl.BlockSpec((1, H), lambda i: (0, 0)),
            pl.BlockSpec((H, H), lambda i: (0, 0)),
            pl.BlockSpec((1, H), lambda i: (0, 0)),
        ],
        out_specs=pl.BlockSpec((_RB, H), lambda i: (i, 0)),
        out_shape=jax.ShapeDtypeStruct((N, H), jnp.float32),
    )(agg, hp, dinv, b2d, wc_pad, bc_pad)


def kernel(x, edge_index, W1, b1, W2, b2, W3, b3, Wc, bc):
    ppt = (EP - E) // NW
    pad_src = jnp.zeros((NW, ppt), jnp.int32)
    pad_dst = jnp.broadcast_to(N + jnp.arange(ppt, dtype=jnp.int32),
                               (NW, ppt))
    src_p = jnp.concatenate([edge_index[0].reshape(NW, E // NW), pad_src], 1)
    dst_p = jnp.concatenate([edge_index[1].reshape(NW, E // NW), pad_dst], 1)
    idx2 = jnp.stack([src_p.reshape(NW, NCHK, CH),
                      dst_p.reshape(NW, NCHK, CH)], axis=2)
    ones_ch = jnp.ones((CH,), jnp.float32)
    zero_n = jnp.zeros((NP,), jnp.float32)
    zrows = jnp.zeros((NP, H), jnp.float32)

    deg_kernel, agg_kernel = _sc_kernels()
    deg = deg_kernel(idx2, ones_ch, zero_n)
    deg3 = deg[:, :N].reshape(NC, N, 1)

    hs1, dinv = _tc_in(deg3, x, W1)
    agg1 = agg_kernel(hs1, idx2, zrows)[:, :N, :]
    hs2 = _tc_mid(agg1, hs1, dinv, b1.reshape(1, H), W2)
    agg2 = agg_kernel(hs2, idx2, zrows)[:, :N, :]
    hs3 = _tc_mid(agg2, hs2, dinv, b2.reshape(1, H), W3)
    agg3 = agg_kernel(hs3, idx2, zrows)[:, :N, :]

    wc_pad = jnp.zeros((H, H), jnp.float32).at[:, :C].set(Wc)
    bc_pad = jnp.zeros((1, H), jnp.float32).at[0, :C].set(bc)
    out = _tc_out(agg3, hs3, dinv, b3.reshape(1, H), wc_pad, bc_pad)
    return out[:, :C]


# speedup vs baseline: 1.5598x; 1.4305x over previous
import functools

import jax
import jax.numpy as jnp
from jax import lax
from jax.experimental import pallas as pl
from jax.experimental.pallas import tpu as pltpu
from jax.experimental.pallas import tpu_sc as plsc

N = 10000
E = 320000
D = 128
H = 128
C = 10

NP = 10240
NC = 2
NS = 16
NW = NC * NS
EPT = E // NW
CH = 80
NCHK = EPT // CH
RPT = NP // NS


def _deg_body(dst_hbm, ones_hbm, zero_hbm, deg_out, dst_v, ones_v, acc_sh):
    cid = lax.axis_index("c")
    sid = lax.axis_index("s")
    wid = cid * NS + sid
    rbase = sid * RPT
    pltpu.sync_copy(zero_hbm.at[pl.ds(rbase, RPT)], acc_sh.at[pl.ds(rbase, RPT)])
    pltpu.sync_copy(ones_hbm, ones_v)
    pltpu.sync_copy(dst_hbm.at[wid], dst_v)
    plsc.subcore_barrier()

    def body(j, carry):
        pltpu.sync_copy(ones_v, acc_sh.at[dst_v.at[j]], add=True)
        return carry

    lax.fori_loop(0, NCHK, body, 0)
    plsc.subcore_barrier()
    pltpu.sync_copy(acc_sh.at[pl.ds(rbase, RPT)],
                    deg_out.at[cid, pl.ds(rbase, RPT)])


def _agg_body(hs_hbm, src_hbm, dst_hbm, zrows_hbm, agg_out,
              src_v, dst_v, gath_v, acc_sh, sem):
    cid = lax.axis_index("c")
    sid = lax.axis_index("s")
    wid = cid * NS + sid
    rbase = sid * RPT
    pltpu.sync_copy(zrows_hbm.at[pl.ds(rbase, RPT)],
                    acc_sh.at[pl.ds(rbase, RPT)])
    pltpu.sync_copy(src_hbm.at[wid], src_v)
    pltpu.sync_copy(dst_hbm.at[wid], dst_v)
    plsc.subcore_barrier()

    def body(j, carry):
        pltpu.async_copy(hs_hbm.at[src_v.at[j]], gath_v, sem).wait()
        pltpu.sync_copy(gath_v, acc_sh.at[dst_v.at[j]], add=True)
        return carry

    lax.fori_loop(0, NCHK, body, 0)
    plsc.subcore_barrier()
    pltpu.sync_copy(acc_sh.at[pl.ds(rbase, RPT), :],
                    agg_out.at[cid, pl.ds(rbase, RPT), :])


@functools.cache
def _sc_kernels():
    mesh = plsc.VectorSubcoreMesh(core_axis_name="c", subcore_axis_name="s")
    deg_k = pl.kernel(
        _deg_body,
        mesh=mesh,
        out_type=jax.ShapeDtypeStruct((NC, NP), jnp.float32),
        scratch_types=[
            pltpu.VMEM((NCHK, CH), jnp.int32),
            pltpu.VMEM((CH,), jnp.float32),
            pltpu.VMEM_SHARED((NP,), jnp.float32),
        ],
    )
    agg_k = pl.kernel(
        _agg_body,
        mesh=mesh,
        out_type=jax.ShapeDtypeStruct((NC, NP, H), jnp.float32),
        scratch_types=[
            pltpu.VMEM((NCHK, CH), jnp.int32),
            pltpu.VMEM((NCHK, CH), jnp.int32),
            pltpu.VMEM((CH, H), jnp.float32),
            pltpu.VMEM_SHARED((NP, H), jnp.float32),
            pltpu.SemaphoreType.DMA,
        ],
    )
    return deg_k, agg_k


_RB = 1000


def _tc_in_body(deg_ref, x_ref, w_ref, hs_ref, dinv_ref):
    dd = deg_ref[0] + deg_ref[1] + 1.0
    dinv = lax.rsqrt(dd)
    h = jnp.dot(x_ref[...], w_ref[...], preferred_element_type=jnp.float32)
    hs_ref[...] = h * dinv
    dinv_ref[...] = dinv


def _tc_mid_body(agg_ref, hp_ref, dinv_ref, b_ref, w_ref, out_ref):
    dinv = dinv_ref[...]
    pre = (agg_ref[0] + agg_ref[1] + hp_ref[...]) * dinv + b_ref[...]
    act = jnp.maximum(pre, 0.0)
    out_ref[...] = jnp.dot(act, w_ref[...],
                           preferred_element_type=jnp.float32) * dinv


def _tc_out_body(agg_ref, hp_ref, dinv_ref, b_ref, wc_ref, bc_ref, out_ref):
    dinv = dinv_ref[...]
    pre = (agg_ref[0] + agg_ref[1] + hp_ref[...]) * dinv + b_ref[...]
    act = jnp.maximum(pre, 0.0)
    out_ref[...] = jnp.dot(act, wc_ref[...],
                           preferred_element_type=jnp.float32) + bc_ref[...]


def _tc_in(deg3, x, w):
    return pl.pallas_call(
        _tc_in_body,
        grid=(N // _RB,),
        in_specs=[
            pl.BlockSpec((NC, _RB, 1), lambda i: (0, i, 0)),
            pl.BlockSpec((_RB, D), lambda i: (i, 0)),
            pl.BlockSpec((D, H), lambda i: (0, 0)),
        ],
        out_specs=[
            pl.BlockSpec((_RB, H), lambda i: (i, 0)),
            pl.BlockSpec((_RB, 1), lambda i: (i, 0)),
        ],
        out_shape=[
            jax.ShapeDtypeStruct((N, H), jnp.float32),
            jax.ShapeDtypeStruct((N, 1), jnp.float32),
        ],
    )(deg3, x, w)


def _tc_mid(agg, hp, dinv, b2d, w):
    return pl.pallas_call(
        _tc_mid_body,
        grid=(N // _RB,),
        in_specs=[
            pl.BlockSpec((NC, _RB, H), lambda i: (0, i, 0)),
            pl.BlockSpec((_RB, H), lambda i: (i, 0)),
            pl.BlockSpec((_RB, 1), lambda i: (i, 0)),
            pl.BlockSpec((1, H), lambda i: (0, 0)),
            pl.BlockSpec((H, H), lambda i: (0, 0)),
        ],
        out_specs=pl.BlockSpec((_RB, H), lambda i: (i, 0)),
        out_shape=jax.ShapeDtypeStruct((N, H), jnp.float32),
    )(agg, hp, dinv, b2d, w)


def _tc_out(agg, hp, dinv, b2d, wc_pad, bc_pad):
    return pl.pallas_call(
        _tc_out_body,
        grid=(N // _RB,),
        in_specs=[
            pl.BlockSpec((NC, _RB, H), lambda i: (0, i, 0)),
            pl.BlockSpec((_RB, H), lambda i: (i, 0)),
            pl.BlockSpec((_RB, 1), lambda i: (i, 0)),
            pl.BlockSpec((1, H), lambda i: (0, 0)),
            pl.BlockSpec((H, H), lambda i: (0, 0)),
            pl.BlockSpec((1, H), lambda i: (0, 0)),
        ],
        out_specs=pl.BlockSpec((_RB, H), lambda i: (i, 0)),
        out_shape=jax.ShapeDtypeStruct((N, H), jnp.float32),
    )(agg, hp, dinv, b2d, wc_pad, bc_pad)


def kernel(x, edge_index, W1, b1, W2, b2, W3, b3, Wc, bc):
    src_r = edge_index[0].reshape(NW, NCHK, CH)
    dst_r = edge_index[1].reshape(NW, NCHK, CH)
    ones_ch = jnp.ones((CH,), jnp.float32)
    zero_n = jnp.zeros((NP,), jnp.float32)
    zrows = jnp.zeros((NP, H), jnp.float32)

    deg_kernel, agg_kernel = _sc_kernels()
    deg = deg_kernel(dst_r, ones_ch, zero_n)
    deg3 = deg[:, :N].reshape(NC, N, 1)

    hs1, dinv = _tc_in(deg3, x, W1)
    agg1 = agg_kernel(hs1, src_r, dst_r, zrows)[:, :N, :]
    hs2 = _tc_mid(agg1, hs1, dinv, b1.reshape(1, H), W2)
    agg2 = agg_kernel(hs2, src_r, dst_r, zrows)[:, :N, :]
    hs3 = _tc_mid(agg2, hs2, dinv, b2.reshape(1, H), W3)
    agg3 = agg_kernel(hs3, src_r, dst_r, zrows)[:, :N, :]

    wc_pad = jnp.zeros((H, H), jnp.float32).at[:, :C].set(Wc)
    bc_pad = jnp.zeros((1, H), jnp.float32).at[0, :C].set(bc)
    out = _tc_out(agg3, hs3, dinv, b3.reshape(1, H), wc_pad, bc_pad)
    return out[:, :C]
